# baseline (speedup 1.0000x reference)
import functools

import jax
import jax.numpy as jnp
import numpy as np
from jax.experimental import pallas as pl
from jax.experimental.pallas import tpu as pltpu
from jax.experimental.pallas import tpu_sc as plsc

N = 10000
E = 160000
D = 128
H = 4
DH = D // H

NEG = -1e30

NC = 2
NS = 16
NW = NC * NS
CH = 40

EBLK = 1280
EGRID = E // EBLK
NBLK = 1000
NGRID = N // NBLK


def _mm(x, w):
    return jax.lax.dot_general(x.astype(jnp.bfloat16), w.astype(jnp.bfloat16),
                               (((1,), (0,)), ((), ())),
                               preferred_element_type=jnp.float32)


def _gelu(x):
    return 0.5 * x * (1.0 + jax.lax.erf(x * 0.7071067811865476))


def _sc_gather(table, idx1d):
    total = idx1d.shape[0]
    rows_per_w = total // NW
    K = 10
    groups = rows_per_w // (K * CH)
    mesh = plsc.VectorSubcoreMesh(core_axis_name="c", subcore_axis_name="s")

    @functools.partial(
        pl.kernel,
        out_type=jax.ShapeDtypeStruct((total, D), jnp.float32),
        mesh=mesh,
        scratch_types=[pltpu.VMEM((CH,), jnp.int32)] * K + [
            pltpu.VMEM((K * CH, D), jnp.float32),
            pltpu.SemaphoreType.DMA,
            pltpu.SemaphoreType.DMA,
        ],
    )
    def k(table_hbm, idx_hbm, out_hbm, *rest):
        idx_bufs = rest[:K]
        rows_v, sem_i, sem_g = rest[K:]
        cid = jax.lax.axis_index("c")
        sid = jax.lax.axis_index("s")
        wid = sid * NC + cid
        base = wid * rows_per_w

        @pl.loop(0, groups)
        def _(g):
            g0 = base + g * K * CH
            descs = [
                pltpu.async_copy(idx_hbm.at[pl.ds(g0 + i * CH, CH)],
                                 idx_bufs[i], sem_i)
                for i in range(K)
            ]
            for d in descs:
                d.wait()
            gds = [
                pltpu.async_copy(table_hbm.at[idx_bufs[i]],
                                 rows_v.at[pl.ds(i * CH, CH)], sem_g)
                for i in range(K)
            ]
            for d in gds:
                d.wait()
            pltpu.sync_copy(rows_v, out_hbm.at[pl.ds(g0, K * CH)])

    return k(table, idx1d)


def _sc_scatter(numv, exb, sidx, zn):
    chunks_per_tile = E // CH // NW
    e_per_tile = E // NW
    RC = 16
    mesh = plsc.VectorSubcoreMesh(core_axis_name="c", subcore_axis_name="s")

    @functools.partial(
        pl.kernel,
        out_type=(jax.ShapeDtypeStruct((2 * N, D), jnp.float32),
                  jax.ShapeDtypeStruct((2 * N, D), jnp.float32)),
        mesh=mesh,
        scratch_types=[pltpu.VMEM((CH,), jnp.int32)] * 5 + [
            pltpu.VMEM((CH, D), jnp.float32)] * 5 + [
            pltpu.VMEM((CH, D), jnp.float32),
            pltpu.VMEM_SHARED((N, D), jnp.float32),
            pltpu.SemaphoreType.DMA,
            pltpu.SemaphoreType.DMA,
        ],
    )
    def k(numv_hbm, exb_hbm, idx_hbm, zn_hbm, onum_hbm, oden_hbm, *rest):
        KS = 5
        idx_bufs = rest[:KS]
        pay_bufs = rest[KS:2 * KS]
        stage_v, acc, sem_l, sem_s = rest[2 * KS:]
        cid = jax.lax.axis_index("c")
        sid = jax.lax.axis_index("s")
        r0 = sid * RC * CH
        wid = cid * NS + sid
        e_base = wid * e_per_tile
        groups = chunks_per_tile // KS

        def zero_acc():
            pltpu.sync_copy(zn_hbm.at[pl.ds(0, CH)], stage_v)

            @pl.loop(0, RC)
            def _(j):
                @pl.when(r0 + j * CH < N)
                def _():
                    pltpu.sync_copy(stage_v, acc.at[pl.ds(r0 + j * CH, CH)])

        def scatter_pass(src_hbm):
            @pl.loop(0, groups)
            def _(g):
                g0 = e_base + g * KS * CH
                lds = [pltpu.async_copy(idx_hbm.at[pl.ds(g0 + i * CH, CH)],
                                        idx_bufs[i], sem_l)
                       for i in range(KS)]
                lds += [pltpu.async_copy(src_hbm.at[pl.ds(g0 + i * CH, CH)],
                                         pay_bufs[i], sem_l)
                        for i in range(KS)]
                for d in lds:
                    d.wait()
                sds = [pltpu.async_copy(pay_bufs[i], acc.at[idx_bufs[i]],
                                        sem_s, add=True)
                       for i in range(KS)]
                for d in sds:
                    d.wait()

        def copy_out(dst_hbm):
            @pl.loop(0, RC)
            def _(j):
                c0 = r0 + j * CH

                @pl.when(c0 < N)
                def _():
                    pltpu.sync_copy(acc.at[pl.ds(c0, CH)], stage_v)
                    pltpu.sync_copy(stage_v, dst_hbm.at[pl.ds(cid * N + c0, CH)])

        zero_acc()
        plsc.subcore_barrier()
        scatter_pass(numv_hbm)
        plsc.subcore_barrier()
        copy_out(onum_hbm)
        plsc.subcore_barrier()
        zero_acc()
        plsc.subcore_barrier()
        scatter_pass(exb_hbm)
        plsc.subcore_barrier()
        copy_out(oden_hbm)

    return k(numv, exb, sidx, zn)


def _row(b):
    return b[0:1, :]


def _tc1_body(hvs, he, hvd, w1a, w1b, w1c, b1, w2, b2, w3, b3, l8_ref, gm_ref):
    i = pl.program_id(0)
    x = _mm(hvs[...], w1a[...]) + _mm(he[...], w1b[...]) + _mm(hvd[...], w1c[...])
    x = jnp.maximum(x + _row(b1[...]), 0.0)
    x = jnp.maximum(_mm(x, w2[...]) + _row(b2[...]), 0.0)
    l8 = _mm(x, w3[...]) + _row(b3[...])
    l8_ref[...] = l8
    bm = jnp.full((8, 128), jnp.max(l8), jnp.float32)

    @pl.when(i == 0)
    def _():
        gm_ref[...] = bm

    @pl.when(i > 0)
    def _():
        gm_ref[...] = jnp.maximum(gm_ref[...], bm)


def _edge_logits(gath, he, w1a, w1b, w1c, b1, w2, b2, w3, b3):
    eb = lambda i: (i, 0)
    eb2 = lambda i: (i + EGRID, 0)
    cb = lambda i: (0, 0)
    return pl.pallas_call(
        _tc1_body,
        grid=(EGRID,),
        in_specs=[
            pl.BlockSpec((EBLK, D), eb),
            pl.BlockSpec((EBLK, D), eb),
            pl.BlockSpec((EBLK, D), eb2),
            pl.BlockSpec((D, D), cb),
            pl.BlockSpec((D, D), cb),
            pl.BlockSpec((D, D), cb),
            pl.BlockSpec((8, D), cb),
            pl.BlockSpec((D, D), cb),
            pl.BlockSpec((8, D), cb),
            pl.BlockSpec((D, 8), cb),
            pl.BlockSpec((8, 8), cb),
        ],
        out_specs=[
            pl.BlockSpec((EBLK, 8), eb),
            pl.BlockSpec((8, 128), cb),
        ],
        out_shape=[
            jax.ShapeDtypeStruct((E, 8), jnp.float32),
            jax.ShapeDtypeStruct((8, 128), jnp.float32),
        ],
    )(gath, he, gath, w1a, w1b, w1c, b1, w2, b2, w3, b3)


def _tc2_body(l8, he, hvd, gm, wva, wvb, bv1, wv2, bv2, wv3, bv3, s128,
              numv_ref, exb_ref):
    m = gm[0, 0]
    ex8 = jnp.exp(l8[...] - m)
    exb = _mm(ex8, s128[...])
    exb_ref[...] = exb
    x = _gelu(_mm(he[...], wva[...]) + _mm(hvd[...], wvb[...]) + _row(bv1[...]))
    x = _gelu(_mm(x, wv2[...]) + _row(bv2[...]))
    v = _mm(x, wv3[...]) + _row(bv3[...])
    numv_ref[...] = exb * v


def _edge_payload(l8, he, gath, gm, wva, wvb, bv1, wv2, bv2, wv3, bv3, s128):
    eb = lambda i: (i, 0)
    eb2 = lambda i: (i + EGRID, 0)
    cb = lambda i: (0, 0)
    return pl.pallas_call(
        _tc2_body,
        grid=(EGRID,),
        in_specs=[
            pl.BlockSpec((EBLK, 8), eb),
            pl.BlockSpec((EBLK, D), eb),
            pl.BlockSpec((EBLK, D), eb2),
            pl.BlockSpec((8, 128), cb),
            pl.BlockSpec((D, D), cb),
            pl.BlockSpec((D, D), cb),
            pl.BlockSpec((8, D), cb),
            pl.BlockSpec((D, D), cb),
            pl.BlockSpec((8, D), cb),
            pl.BlockSpec((D, D), cb),
            pl.BlockSpec((8, D), cb),
            pl.BlockSpec((8, 128), cb),
        ],
        out_specs=[
            pl.BlockSpec((EBLK, D), eb),
            pl.BlockSpec((EBLK, D), eb),
        ],
        out_shape=[
            jax.ShapeDtypeStruct((E, D), jnp.float32),
            jax.ShapeDtypeStruct((E, D), jnp.float32),
        ],
    )(l8, he, gath, gm, wva, wvb, bv1, wv2, bv2, wv3, bv3, s128)


def _tc3_body(n0, n1, d0, d1, hv, wot, x1_ref, s_ref, q_ref):
    i = pl.program_id(0)
    num = n0[...] + n1[...]
    dden = d0[...] + d1[...]
    pos = dden > 0.0
    hagg = jnp.where(pos, num, 0.0) / jnp.where(pos, dden, 1.0)
    x1 = hv[...] + _mm(hagg, wot[...])
    x1_ref[...] = x1
    s = jnp.broadcast_to(jnp.sum(x1, axis=0)[None, :], (8, 128))
    q = jnp.broadcast_to(jnp.sum(x1 * x1, axis=0)[None, :], (8, 128))

    @pl.when(i == 0)
    def _():
        s_ref[...] = s
        q_ref[...] = q

    @pl.when(i > 0)
    def _():
        s_ref[...] += s
        q_ref[...] += q


def _node_agg(onum, oden, hv, wot):
    nb = lambda i: (i, 0)
    nb2 = lambda i: (i + NGRID, 0)
    cb = lambda i: (0, 0)
    return pl.pallas_call(
        _tc3_body,
        grid=(NGRID,),
        in_specs=[
            pl.BlockSpec((NBLK, D), nb),
            pl.BlockSpec((NBLK, D), nb2),
            pl.BlockSpec((NBLK, D), nb),
            pl.BlockSpec((NBLK, D), nb2),
            pl.BlockSpec((NBLK, D), nb),
            pl.BlockSpec((D, D), cb),
        ],
        out_specs=[
            pl.BlockSpec((NBLK, D), nb),
            pl.BlockSpec((8, 128), cb),
            pl.BlockSpec((8, 128), cb),
        ],
        out_shape=[
            jax.ShapeDtypeStruct((N, D), jnp.float32),
            jax.ShapeDtypeStruct((8, 128), jnp.float32),
            jax.ShapeDtypeStruct((8, 128), jnp.float32),
        ],
    )(onum, onum, oden, oden, hv, wot)


def _tc4_body(x1, s1, q1, g0, b0, w1t, bb1, w2t, bb2, x2_ref, s_ref, q_ref):
    i = pl.program_id(0)
    m = _row(s1[...]) * (1.0 / N)
    v = _row(q1[...]) * (1.0 / N) - m * m
    inv = jax.lax.rsqrt(v + 1e-5)
    xn = (x1[...] - m) * inv * _row(g0[...]) + _row(b0[...])
    h = jnp.maximum(_mm(xn, w1t[...]) + _row(bb1[...]), 0.0)
    x2 = xn + _mm(h, w2t[...]) + _row(bb2[...])
    x2_ref[...] = x2
    s = jnp.broadcast_to(jnp.sum(x2, axis=0)[None, :], (8, 128))
    q = jnp.broadcast_to(jnp.sum(x2 * x2, axis=0)[None, :], (8, 128))

    @pl.when(i == 0)
    def _():
        s_ref[...] = s
        q_ref[...] = q

    @pl.when(i > 0)
    def _():
        s_ref[...] += s
        q_ref[...] += q


def _node_dense(x1, s1, q1, g0, b0, w1t, bb1, w2t, bb2):
    nb = lambda i: (i, 0)
    cb = lambda i: (0, 0)
    return pl.pallas_call(
        _tc4_body,
        grid=(NGRID,),
        in_specs=[
            pl.BlockSpec((NBLK, D), nb),
            pl.BlockSpec((8, 128), cb),
            pl.BlockSpec((8, 128), cb),
            pl.BlockSpec((8, D), cb),
            pl.BlockSpec((8, D), cb),
            pl.BlockSpec((D, 4 * D), cb),
            pl.BlockSpec((8, 4 * D), cb),
            pl.BlockSpec((4 * D, D), cb),
            pl.BlockSpec((8, D), cb),
        ],
        out_specs=[
            pl.BlockSpec((NBLK, D), nb),
            pl.BlockSpec((8, 128), cb),
            pl.BlockSpec((8, 128), cb),
        ],
        out_shape=[
            jax.ShapeDtypeStruct((N, D), jnp.float32),
            jax.ShapeDtypeStruct((8, 128), jnp.float32),
            jax.ShapeDtypeStruct((8, 128), jnp.float32),
        ],
    )(x1, s1, q1, g0, b0, w1t, bb1, w2t, bb2)


def _bn_body(nrows, x, s, q, g, b, out_ref):
    m = _row(s[...]) * (1.0 / nrows)
    v = _row(q[...]) * (1.0 / nrows) - m * m
    inv = jax.lax.rsqrt(v + 1e-5)
    out_ref[...] = (x[...] - m) * inv * _row(g[...]) + _row(b[...])


def _bn_apply(x, s, q, g, b, blk):
    nrows, _ = x.shape
    nb = lambda i: (i, 0)
    cb = lambda i: (0, 0)
    return pl.pallas_call(
        functools.partial(_bn_body, nrows),
        grid=(nrows // blk,),
        in_specs=[
            pl.BlockSpec((blk, D), nb),
            pl.BlockSpec((8, 128), cb),
            pl.BlockSpec((8, 128), cb),
            pl.BlockSpec((8, D), cb),
            pl.BlockSpec((8, D), cb),
        ],
        out_specs=pl.BlockSpec((blk, D), nb),
        out_shape=jax.ShapeDtypeStruct((nrows, D), jnp.float32),
    )(x, s, q, g, b)


def _tc6_body(hvs, he, hvd, w1a, w1b, w1c, b1, w2, b2, w3, b3,
              xe_ref, s_ref, q_ref):
    i = pl.program_id(0)
    x = _mm(hvs[...], w1a[...]) + _mm(he[...], w1b[...]) + _mm(hvd[...], w1c[...])
    x = _gelu(x + _row(b1[...]))
    x = _gelu(_mm(x, w2[...]) + _row(b2[...]))
    msg = _mm(x, w3[...]) + _row(b3[...])
    xe = he[...] + msg
    xe_ref[...] = xe
    s = jnp.broadcast_to(jnp.sum(xe, axis=0)[None, :], (8, 128))
    q = jnp.broadcast_to(jnp.sum(xe * xe, axis=0)[None, :], (8, 128))

    @pl.when(i == 0)
    def _():
        s_ref[...] = s
        q_ref[...] = q

    @pl.when(i > 0)
    def _():
        s_ref[...] += s
        q_ref[...] += q


def _edge_mlp(gath, he, w1a, w1b, w1c, b1, w2, b2, w3, b3):
    eb = lambda i: (i, 0)
    eb2 = lambda i: (i + EGRID, 0)
    cb = lambda i: (0, 0)
    return pl.pallas_call(
        _tc6_body,
        grid=(EGRID,),
        in_specs=[
            pl.BlockSpec((EBLK, D), eb),
            pl.BlockSpec((EBLK, D), eb),
            pl.BlockSpec((EBLK, D), eb2),
            pl.BlockSpec((D, D), cb),
            pl.BlockSpec((D, D), cb),
            pl.BlockSpec((D, D), cb),
            pl.BlockSpec((8, D), cb),
            pl.BlockSpec((D, D), cb),
            pl.BlockSpec((8, D), cb),
            pl.BlockSpec((D, D), cb),
            pl.BlockSpec((8, D), cb),
        ],
        out_specs=[
            pl.BlockSpec((EBLK, D), eb),
            pl.BlockSpec((8, 128), cb),
            pl.BlockSpec((8, 128), cb),
        ],
        out_shape=[
            jax.ShapeDtypeStruct((E, D), jnp.float32),
            jax.ShapeDtypeStruct((8, 128), jnp.float32),
            jax.ShapeDtypeStruct((8, 128), jnp.float32),
        ],
    )(gath, he, gath, w1a, w1b, w1c, b1, w2, b2, w3, b3)


def _bc8(b):
    return jnp.broadcast_to(b[None, :], (8, b.shape[0])).astype(jnp.float32)


def kernel(h_V, h_E, edge_idx, batch_id, params):
    p = params
    src = edge_idx[0]
    dst = edge_idx[1]

    allidx = jnp.concatenate([src, dst])
    gath = _sc_gather(h_V, allidx)

    sc = 1.0 / np.sqrt(DH)
    w3p = jnp.zeros((D, 8), jnp.float32).at[:, :H].set(p['bias_w3'].T * sc)
    b3p = jnp.full((8,), NEG, jnp.float32).at[:H].set(p['bias_b3'] * sc)
    l8, gm = _edge_logits(
        gath, h_E,
        p['bias_w1'][:, :D].T, p['bias_w1'][:, D:2 * D].T, p['bias_w1'][:, 2 * D:].T,
        _bc8(p['bias_b1']), p['bias_w2'].T, _bc8(p['bias_b2']), w3p, _bc8(b3p))

    s128 = np.zeros((8, 128), np.float32)
    for h in range(H):
        s128[h, h * DH:(h + 1) * DH] = 1.0
    numv, exb = _edge_payload(
        l8, h_E, gath, gm,
        p['wv_w1'][:, :D].T, p['wv_w1'][:, D:].T, _bc8(p['wv_b1']),
        p['wv_w2'].T, _bc8(p['wv_b2']), p['wv_w3'].T, _bc8(p['wv_b3']),
        jnp.asarray(s128))

    onum, oden = _sc_scatter(numv, exb, src, jnp.zeros((N, D), jnp.float32))

    x1, s1, q1 = _node_agg(onum, oden, h_V, p['wo'].T)
    x2, s2, q2 = _node_dense(x1, s1, q1, _bc8(p['bn0_g']), _bc8(p['bn0_b']),
                             p['dense_w1'].T, _bc8(p['dense_b1']),
                             p['dense_w2'].T, _bc8(p['dense_b2']))
    h_V2 = _bn_apply(x2, s2, q2, _bc8(p['bn1_g']), _bc8(p['bn1_b']), NBLK)

    gath2 = _sc_gather(h_V2, allidx)
    xe, se, qe = _edge_mlp(
        gath2, h_E,
        p['e_w11'][:, :D].T, p['e_w11'][:, D:2 * D].T, p['e_w11'][:, 2 * D:].T,
        _bc8(p['e_b11']), p['e_w12'].T, _bc8(p['e_b12']),
        p['e_w13'].T, _bc8(p['e_b13']))
    h_E2 = _bn_apply(xe, se, qe, _bc8(p['bne_g']), _bc8(p['bne_b']), EBLK)

    return h_V2, h_E2


# speedup vs baseline: 1.0129x; 1.0129x over previous
import functools

import jax
import jax.numpy as jnp
import numpy as np
from jax.experimental import pallas as pl
from jax.experimental.pallas import tpu as pltpu
from jax.experimental.pallas import tpu_sc as plsc

N = 10000
E = 160000
D = 128
H = 4
DH = D // H

NEG = -1e30

NC = 2
NS = 16
NW = NC * NS
CH = 40

EBLK = 1280
EGRID = E // EBLK
NBLK = 1000
NGRID = N // NBLK


def _mm(x, w):
    return jax.lax.dot_general(x.astype(jnp.bfloat16), w.astype(jnp.bfloat16),
                               (((1,), (0,)), ((), ())),
                               preferred_element_type=jnp.float32)


def _gelu(x):
    return 0.5 * x * (1.0 + jax.lax.erf(x * 0.7071067811865476))


def _sc_gather(table, idx1d):
    total = idx1d.shape[0]
    rows_per_w = total // NW
    K = 10
    groups = rows_per_w // (K * CH)
    mesh = plsc.VectorSubcoreMesh(core_axis_name="c", subcore_axis_name="s")

    @functools.partial(
        pl.kernel,
        out_type=jax.ShapeDtypeStruct((total, D), jnp.float32),
        mesh=mesh,
        scratch_types=[pltpu.VMEM((CH,), jnp.int32)] * K + [
            pltpu.VMEM((K * CH, D), jnp.float32),
            pltpu.SemaphoreType.DMA,
            pltpu.SemaphoreType.DMA,
        ],
    )
    def k(table_hbm, idx_hbm, out_hbm, *rest):
        idx_bufs = rest[:K]
        rows_v, sem_i, sem_g = rest[K:]
        cid = jax.lax.axis_index("c")
        sid = jax.lax.axis_index("s")
        wid = sid * NC + cid
        base = wid * rows_per_w

        @pl.loop(0, groups)
        def _(g):
            g0 = base + g * K * CH
            descs = [
                pltpu.async_copy(idx_hbm.at[pl.ds(g0 + i * CH, CH)],
                                 idx_bufs[i], sem_i)
                for i in range(K)
            ]
            for d in descs:
                d.wait()
            gds = [
                pltpu.async_copy(table_hbm.at[idx_bufs[i]],
                                 rows_v.at[pl.ds(i * CH, CH)], sem_g)
                for i in range(K)
            ]
            for d in gds:
                d.wait()
            pltpu.sync_copy(rows_v, out_hbm.at[pl.ds(g0, K * CH)])

    return k(table, idx1d)


def _sc_scatter(numv, exb, sidx, zn):
    e_per_tile = E // NS
    chunks_per_tile = e_per_tile // CH
    RC = 16
    KS = 5
    mesh = plsc.VectorSubcoreMesh(core_axis_name="c", subcore_axis_name="s")

    @functools.partial(
        pl.kernel,
        out_type=(jax.ShapeDtypeStruct((N, D), jnp.float32),
                  jax.ShapeDtypeStruct((N, D), jnp.float32)),
        mesh=mesh,
        scratch_types=[pltpu.VMEM((CH,), jnp.int32)] * KS + [
            pltpu.VMEM((CH, D), jnp.float32)] * KS + [
            pltpu.VMEM((CH, D), jnp.float32),
            pltpu.VMEM_SHARED((N, D), jnp.float32),
            pltpu.SemaphoreType.DMA,
            pltpu.SemaphoreType.DMA,
        ],
    )
    def k(numv_hbm, exb_hbm, idx_hbm, zn_hbm, onum_hbm, oden_hbm, *rest):
        idx_bufs = rest[:KS]
        pay_bufs = rest[KS:2 * KS]
        stage_v, acc, sem_l, sem_s = rest[2 * KS:]
        cid = jax.lax.axis_index("c")
        sid = jax.lax.axis_index("s")
        r0 = sid * RC * CH
        e_base = sid * e_per_tile
        groups = chunks_per_tile // KS

        pltpu.sync_copy(zn_hbm.at[pl.ds(0, CH)], stage_v)

        @pl.loop(0, RC)
        def _(j):
            @pl.when(r0 + j * CH < N)
            def _():
                pltpu.sync_copy(stage_v, acc.at[pl.ds(r0 + j * CH, CH)])

        plsc.subcore_barrier()

        def scatter_pass(src_hbm):
            @pl.loop(0, groups)
            def _(g):
                g0 = e_base + g * KS * CH
                lds = [pltpu.async_copy(idx_hbm.at[pl.ds(g0 + i * CH, CH)],
                                        idx_bufs[i], sem_l)
                       for i in range(KS)]
                lds += [pltpu.async_copy(src_hbm.at[pl.ds(g0 + i * CH, CH)],
                                         pay_bufs[i], sem_l)
                        for i in range(KS)]
                for d in lds:
                    d.wait()
                sds = [pltpu.async_copy(pay_bufs[i], acc.at[idx_bufs[i]],
                                        sem_s, add=True)
                       for i in range(KS)]
                for d in sds:
                    d.wait()

        @pl.when(cid == 0)
        def _():
            scatter_pass(numv_hbm)

        @pl.when(cid == 1)
        def _():
            scatter_pass(exb_hbm)

        plsc.subcore_barrier()

        def copy_out(dst_hbm):
            @pl.loop(0, RC)
            def _(j):
                c0 = r0 + j * CH

                @pl.when(c0 < N)
                def _():
                    pltpu.sync_copy(acc.at[pl.ds(c0, CH)], stage_v)
                    pltpu.sync_copy(stage_v, dst_hbm.at[pl.ds(c0, CH)])

        @pl.when(cid == 0)
        def _():
            copy_out(onum_hbm)

        @pl.when(cid == 1)
        def _():
            copy_out(oden_hbm)

    return k(numv, exb, sidx, zn)


def _row(b):
    return b[0:1, :]


def _tc1_body(hvs, he, hvd, w1a, w1b, w1c, b1, w2, b2, w3, b3, l8_ref, gm_ref):
    i = pl.program_id(0)
    x = _mm(hvs[...], w1a[...]) + _mm(he[...], w1b[...]) + _mm(hvd[...], w1c[...])
    x = jnp.maximum(x + _row(b1[...]), 0.0)
    x = jnp.maximum(_mm(x, w2[...]) + _row(b2[...]), 0.0)
    l8 = _mm(x, w3[...]) + _row(b3[...])
    l8_ref[...] = l8
    bm = jnp.full((8, 128), jnp.max(l8), jnp.float32)

    @pl.when(i == 0)
    def _():
        gm_ref[...] = bm

    @pl.when(i > 0)
    def _():
        gm_ref[...] = jnp.maximum(gm_ref[...], bm)


def _edge_logits(gath, he, w1a, w1b, w1c, b1, w2, b2, w3, b3):
    eb = lambda i: (i, 0)
    eb2 = lambda i: (i + EGRID, 0)
    cb = lambda i: (0, 0)
    return pl.pallas_call(
        _tc1_body,
        grid=(EGRID,),
        in_specs=[
            pl.BlockSpec((EBLK, D), eb),
            pl.BlockSpec((EBLK, D), eb),
            pl.BlockSpec((EBLK, D), eb2),
            pl.BlockSpec((D, D), cb),
            pl.BlockSpec((D, D), cb),
            pl.BlockSpec((D, D), cb),
            pl.BlockSpec((8, D), cb),
            pl.BlockSpec((D, D), cb),
            pl.BlockSpec((8, D), cb),
            pl.BlockSpec((D, 8), cb),
            pl.BlockSpec((8, 8), cb),
        ],
        out_specs=[
            pl.BlockSpec((EBLK, 8), eb),
            pl.BlockSpec((8, 128), cb),
        ],
        out_shape=[
            jax.ShapeDtypeStruct((E, 8), jnp.float32),
            jax.ShapeDtypeStruct((8, 128), jnp.float32),
        ],
    )(gath, he, gath, w1a, w1b, w1c, b1, w2, b2, w3, b3)


def _tc2_body(l8, he, hvd, gm, wva, wvb, bv1, wv2, bv2, wv3, bv3, s128,
              numv_ref, exb_ref):
    m = gm[0, 0]
    ex8 = jnp.exp(l8[...] - m)
    exb = _mm(ex8, s128[...])
    exb_ref[...] = exb
    x = _gelu(_mm(he[...], wva[...]) + _mm(hvd[...], wvb[...]) + _row(bv1[...]))
    x = _gelu(_mm(x, wv2[...]) + _row(bv2[...]))
    v = _mm(x, wv3[...]) + _row(bv3[...])
    numv_ref[...] = exb * v


def _edge_payload(l8, he, gath, gm, wva, wvb, bv1, wv2, bv2, wv3, bv3, s128):
    eb = lambda i: (i, 0)
    eb2 = lambda i: (i + EGRID, 0)
    cb = lambda i: (0, 0)
    return pl.pallas_call(
        _tc2_body,
        grid=(EGRID,),
        in_specs=[
            pl.BlockSpec((EBLK, 8), eb),
            pl.BlockSpec((EBLK, D), eb),
            pl.BlockSpec((EBLK, D), eb2),
            pl.BlockSpec((8, 128), cb),
            pl.BlockSpec((D, D), cb),
            pl.BlockSpec((D, D), cb),
            pl.BlockSpec((8, D), cb),
            pl.BlockSpec((D, D), cb),
            pl.BlockSpec((8, D), cb),
            pl.BlockSpec((D, D), cb),
            pl.BlockSpec((8, D), cb),
            pl.BlockSpec((8, 128), cb),
        ],
        out_specs=[
            pl.BlockSpec((EBLK, D), eb),
            pl.BlockSpec((EBLK, D), eb),
        ],
        out_shape=[
            jax.ShapeDtypeStruct((E, D), jnp.float32),
            jax.ShapeDtypeStruct((E, D), jnp.float32),
        ],
    )(l8, he, gath, gm, wva, wvb, bv1, wv2, bv2, wv3, bv3, s128)


def _tc3_body(n0, d0, hv, wot, x1_ref, s_ref, q_ref):
    i = pl.program_id(0)
    num = n0[...]
    dden = d0[...]
    pos = dden > 0.0
    hagg = jnp.where(pos, num, 0.0) / jnp.where(pos, dden, 1.0)
    x1 = hv[...] + _mm(hagg, wot[...])
    x1_ref[...] = x1
    s = jnp.broadcast_to(jnp.sum(x1, axis=0)[None, :], (8, 128))
    q = jnp.broadcast_to(jnp.sum(x1 * x1, axis=0)[None, :], (8, 128))

    @pl.when(i == 0)
    def _():
        s_ref[...] = s
        q_ref[...] = q

    @pl.when(i > 0)
    def _():
        s_ref[...] += s
        q_ref[...] += q


def _node_agg(onum, oden, hv, wot):
    nb = lambda i: (i, 0)
    cb = lambda i: (0, 0)
    return pl.pallas_call(
        _tc3_body,
        grid=(NGRID,),
        in_specs=[
            pl.BlockSpec((NBLK, D), nb),
            pl.BlockSpec((NBLK, D), nb),
            pl.BlockSpec((NBLK, D), nb),
            pl.BlockSpec((D, D), cb),
        ],
        out_specs=[
            pl.BlockSpec((NBLK, D), nb),
            pl.BlockSpec((8, 128), cb),
            pl.BlockSpec((8, 128), cb),
        ],
        out_shape=[
            jax.ShapeDtypeStruct((N, D), jnp.float32),
            jax.ShapeDtypeStruct((8, 128), jnp.float32),
            jax.ShapeDtypeStruct((8, 128), jnp.float32),
        ],
    )(onum, oden, hv, wot)


def _tc4_body(x1, s1, q1, g0, b0, w1t, bb1, w2t, bb2, x2_ref, s_ref, q_ref):
    i = pl.program_id(0)
    m = _row(s1[...]) * (1.0 / N)
    v = _row(q1[...]) * (1.0 / N) - m * m
    inv = jax.lax.rsqrt(v + 1e-5)
    xn = (x1[...] - m) * inv * _row(g0[...]) + _row(b0[...])
    h = jnp.maximum(_mm(xn, w1t[...]) + _row(bb1[...]), 0.0)
    x2 = xn + _mm(h, w2t[...]) + _row(bb2[...])
    x2_ref[...] = x2
    s = jnp.broadcast_to(jnp.sum(x2, axis=0)[None, :], (8, 128))
    q = jnp.broadcast_to(jnp.sum(x2 * x2, axis=0)[None, :], (8, 128))

    @pl.when(i == 0)
    def _():
        s_ref[...] = s
        q_ref[...] = q

    @pl.when(i > 0)
    def _():
        s_ref[...] += s
        q_ref[...] += q


def _node_dense(x1, s1, q1, g0, b0, w1t, bb1, w2t, bb2):
    nb = lambda i: (i, 0)
    cb = lambda i: (0, 0)
    return pl.pallas_call(
        _tc4_body,
        grid=(NGRID,),
        in_specs=[
            pl.BlockSpec((NBLK, D), nb),
            pl.BlockSpec((8, 128), cb),
            pl.BlockSpec((8, 128), cb),
            pl.BlockSpec((8, D), cb),
            pl.BlockSpec((8, D), cb),
            pl.BlockSpec((D, 4 * D), cb),
            pl.BlockSpec((8, 4 * D), cb),
            pl.BlockSpec((4 * D, D), cb),
            pl.BlockSpec((8, D), cb),
        ],
        out_specs=[
            pl.BlockSpec((NBLK, D), nb),
            pl.BlockSpec((8, 128), cb),
            pl.BlockSpec((8, 128), cb),
        ],
        out_shape=[
            jax.ShapeDtypeStruct((N, D), jnp.float32),
            jax.ShapeDtypeStruct((8, 128), jnp.float32),
            jax.ShapeDtypeStruct((8, 128), jnp.float32),
        ],
    )(x1, s1, q1, g0, b0, w1t, bb1, w2t, bb2)


def _bn_body(nrows, x, s, q, g, b, out_ref):
    m = _row(s[...]) * (1.0 / nrows)
    v = _row(q[...]) * (1.0 / nrows) - m * m
    inv = jax.lax.rsqrt(v + 1e-5)
    out_ref[...] = (x[...] - m) * inv * _row(g[...]) + _row(b[...])


def _bn_apply(x, s, q, g, b, blk):
    nrows, _ = x.shape
    nb = lambda i: (i, 0)
    cb = lambda i: (0, 0)
    return pl.pallas_call(
        functools.partial(_bn_body, nrows),
        grid=(nrows // blk,),
        in_specs=[
            pl.BlockSpec((blk, D), nb),
            pl.BlockSpec((8, 128), cb),
            pl.BlockSpec((8, 128), cb),
            pl.BlockSpec((8, D), cb),
            pl.BlockSpec((8, D), cb),
        ],
        out_specs=pl.BlockSpec((blk, D), nb),
        out_shape=jax.ShapeDtypeStruct((nrows, D), jnp.float32),
    )(x, s, q, g, b)


def _tc6_body(hvs, he, hvd, w1a, w1b, w1c, b1, w2, b2, w3, b3,
              xe_ref, s_ref, q_ref):
    i = pl.program_id(0)
    x = _mm(hvs[...], w1a[...]) + _mm(he[...], w1b[...]) + _mm(hvd[...], w1c[...])
    x = _gelu(x + _row(b1[...]))
    x = _gelu(_mm(x, w2[...]) + _row(b2[...]))
    msg = _mm(x, w3[...]) + _row(b3[...])
    xe = he[...] + msg
    xe_ref[...] = xe
    s = jnp.broadcast_to(jnp.sum(xe, axis=0)[None, :], (8, 128))
    q = jnp.broadcast_to(jnp.sum(xe * xe, axis=0)[None, :], (8, 128))

    @pl.when(i == 0)
    def _():
        s_ref[...] = s
        q_ref[...] = q

    @pl.when(i > 0)
    def _():
        s_ref[...] += s
        q_ref[...] += q


def _edge_mlp(gath, he, w1a, w1b, w1c, b1, w2, b2, w3, b3):
    eb = lambda i: (i, 0)
    eb2 = lambda i: (i + EGRID, 0)
    cb = lambda i: (0, 0)
    return pl.pallas_call(
        _tc6_body,
        grid=(EGRID,),
        in_specs=[
            pl.BlockSpec((EBLK, D), eb),
            pl.BlockSpec((EBLK, D), eb),
            pl.BlockSpec((EBLK, D), eb2),
            pl.BlockSpec((D, D), cb),
            pl.BlockSpec((D, D), cb),
            pl.BlockSpec((D, D), cb),
            pl.BlockSpec((8, D), cb),
            pl.BlockSpec((D, D), cb),
            pl.BlockSpec((8, D), cb),
            pl.BlockSpec((D, D), cb),
            pl.BlockSpec((8, D), cb),
        ],
        out_specs=[
            pl.BlockSpec((EBLK, D), eb),
            pl.BlockSpec((8, 128), cb),
            pl.BlockSpec((8, 128), cb),
        ],
        out_shape=[
            jax.ShapeDtypeStruct((E, D), jnp.float32),
            jax.ShapeDtypeStruct((8, 128), jnp.float32),
            jax.ShapeDtypeStruct((8, 128), jnp.float32),
        ],
    )(gath, he, gath, w1a, w1b, w1c, b1, w2, b2, w3, b3)


def _bc8(b):
    return jnp.broadcast_to(b[None, :], (8, b.shape[0])).astype(jnp.float32)


def kernel(h_V, h_E, edge_idx, batch_id, params):
    p = params
    src = edge_idx[0]
    dst = edge_idx[1]

    allidx = jnp.concatenate([src, dst])
    gath = _sc_gather(h_V, allidx)

    sc = 1.0 / np.sqrt(DH)
    w3p = jnp.zeros((D, 8), jnp.float32).at[:, :H].set(p['bias_w3'].T * sc)
    b3p = jnp.full((8,), NEG, jnp.float32).at[:H].set(p['bias_b3'] * sc)
    l8, gm = _edge_logits(
        gath, h_E,
        p['bias_w1'][:, :D].T, p['bias_w1'][:, D:2 * D].T, p['bias_w1'][:, 2 * D:].T,
        _bc8(p['bias_b1']), p['bias_w2'].T, _bc8(p['bias_b2']), w3p, _bc8(b3p))

    s128 = np.zeros((8, 128), np.float32)
    for h in range(H):
        s128[h, h * DH:(h + 1) * DH] = 1.0
    numv, exb = _edge_payload(
        l8, h_E, gath, gm,
        p['wv_w1'][:, :D].T, p['wv_w1'][:, D:].T, _bc8(p['wv_b1']),
        p['wv_w2'].T, _bc8(p['wv_b2']), p['wv_w3'].T, _bc8(p['wv_b3']),
        jnp.asarray(s128))

    onum, oden = _sc_scatter(numv, exb, src, jnp.zeros((N, D), jnp.float32))

    x1, s1, q1 = _node_agg(onum, oden, h_V, p['wo'].T)
    x2, s2, q2 = _node_dense(x1, s1, q1, _bc8(p['bn0_g']), _bc8(p['bn0_b']),
                             p['dense_w1'].T, _bc8(p['dense_b1']),
                             p['dense_w2'].T, _bc8(p['dense_b2']))
    h_V2 = _bn_apply(x2, s2, q2, _bc8(p['bn1_g']), _bc8(p['bn1_b']), NBLK)

    gath2 = _sc_gather(h_V2, allidx)
    xe, se, qe = _edge_mlp(
        gath2, h_E,
        p['e_w11'][:, :D].T, p['e_w11'][:, D:2 * D].T, p['e_w11'][:, 2 * D:].T,
        _bc8(p['e_b11']), p['e_w12'].T, _bc8(p['e_b12']),
        p['e_w13'].T, _bc8(p['e_b13']))
    h_E2 = _bn_apply(xe, se, qe, _bc8(p['bne_g']), _bc8(p['bne_b']), EBLK)

    return h_V2, h_E2


# speedup vs baseline: 1.0292x; 1.0161x over previous
import functools

import jax
import jax.numpy as jnp
import numpy as np
from jax.experimental import pallas as pl
from jax.experimental.pallas import tpu as pltpu
from jax.experimental.pallas import tpu_sc as plsc

N = 10000
E = 160000
D = 128
H = 4
DH = D // H

NEG = -1e30

NC = 2
NS = 16
NW = NC * NS
CH = 40

EBLK = 1280
EGRID = E // EBLK
NBLK = 1000
NGRID = N // NBLK


def _mm(x, w):
    return jax.lax.dot_general(x.astype(jnp.bfloat16), w.astype(jnp.bfloat16),
                               (((1,), (0,)), ((), ())),
                               preferred_element_type=jnp.float32)


def _gelu(x):
    return 0.5 * x * (1.0 + jax.lax.erf(x * 0.7071067811865476))


def _sc_gather(table, idx1d):
    total = idx1d.shape[0]
    rows_per_w = total // NW
    K = 10
    groups = rows_per_w // (K * CH)
    mesh = plsc.VectorSubcoreMesh(core_axis_name="c", subcore_axis_name="s")

    @functools.partial(
        pl.kernel,
        out_type=jax.ShapeDtypeStruct((total, D), jnp.float32),
        mesh=mesh,
        scratch_types=[pltpu.VMEM((CH,), jnp.int32)] * K + [
            pltpu.VMEM((K * CH, D), jnp.float32),
            pltpu.SemaphoreType.DMA,
            pltpu.SemaphoreType.DMA,
        ],
    )
    def k(table_hbm, idx_hbm, out_hbm, *rest):
        idx_bufs = rest[:K]
        rows_v, sem_i, sem_g = rest[K:]
        cid = jax.lax.axis_index("c")
        sid = jax.lax.axis_index("s")
        wid = sid * NC + cid
        base = wid * rows_per_w

        @pl.loop(0, groups)
        def _(g):
            g0 = base + g * K * CH
            descs = [
                pltpu.async_copy(idx_hbm.at[pl.ds(g0 + i * CH, CH)],
                                 idx_bufs[i], sem_i)
                for i in range(K)
            ]
            for d in descs:
                d.wait()
            gds = [
                pltpu.async_copy(table_hbm.at[idx_bufs[i]],
                                 rows_v.at[pl.ds(i * CH, CH)], sem_g)
                for i in range(K)
            ]
            for d in gds:
                d.wait()
            pltpu.sync_copy(rows_v, out_hbm.at[pl.ds(g0, K * CH)])

    return k(table, idx1d)


def _sc_scatter(numv, exb, sidx, zn):
    e_per_tile = E // NS
    chunks_per_tile = e_per_tile // CH
    RC = 16
    KS = 5
    mesh = plsc.VectorSubcoreMesh(core_axis_name="c", subcore_axis_name="s")

    @functools.partial(
        pl.kernel,
        out_type=(jax.ShapeDtypeStruct((N, D), jnp.float32),
                  jax.ShapeDtypeStruct((N, D), jnp.float32)),
        mesh=mesh,
        scratch_types=[pltpu.VMEM((CH,), jnp.int32)] * KS + [
            pltpu.VMEM((CH, D), jnp.float32)] * KS + [
            pltpu.VMEM((CH, D), jnp.float32),
            pltpu.VMEM_SHARED((N, D), jnp.float32),
            pltpu.SemaphoreType.DMA,
            pltpu.SemaphoreType.DMA,
        ],
    )
    def k(numv_hbm, exb_hbm, idx_hbm, zn_hbm, onum_hbm, oden_hbm, *rest):
        idx_bufs = rest[:KS]
        pay_bufs = rest[KS:2 * KS]
        stage_v, acc, sem_l, sem_s = rest[2 * KS:]
        cid = jax.lax.axis_index("c")
        sid = jax.lax.axis_index("s")
        r0 = sid * RC * CH
        e_base = sid * e_per_tile
        groups = chunks_per_tile // KS

        pltpu.sync_copy(zn_hbm.at[pl.ds(0, CH)], stage_v)

        @pl.loop(0, RC)
        def _(j):
            @pl.when(r0 + j * CH < N)
            def _():
                pltpu.sync_copy(stage_v, acc.at[pl.ds(r0 + j * CH, CH)])

        plsc.subcore_barrier()

        def scatter_pass(src_hbm):
            @pl.loop(0, groups)
            def _(g):
                g0 = e_base + g * KS * CH
                lds = [pltpu.async_copy(idx_hbm.at[pl.ds(g0 + i * CH, CH)],
                                        idx_bufs[i], sem_l)
                       for i in range(KS)]
                lds += [pltpu.async_copy(src_hbm.at[pl.ds(g0 + i * CH, CH)],
                                         pay_bufs[i], sem_l)
                        for i in range(KS)]
                for d in lds:
                    d.wait()
                sds = [pltpu.async_copy(pay_bufs[i], acc.at[idx_bufs[i]],
                                        sem_s, add=True)
                       for i in range(KS)]
                for d in sds:
                    d.wait()

        @pl.when(cid == 0)
        def _():
            scatter_pass(numv_hbm)

        @pl.when(cid == 1)
        def _():
            scatter_pass(exb_hbm)

        plsc.subcore_barrier()

        def copy_out(dst_hbm):
            @pl.loop(0, RC)
            def _(j):
                c0 = r0 + j * CH

                @pl.when(c0 < N)
                def _():
                    pltpu.sync_copy(acc.at[pl.ds(c0, CH)], stage_v)
                    pltpu.sync_copy(stage_v, dst_hbm.at[pl.ds(c0, CH)])

        @pl.when(cid == 0)
        def _():
            copy_out(onum_hbm)

        @pl.when(cid == 1)
        def _():
            copy_out(oden_hbm)

    return k(numv, exb, sidx, zn)


def _row(b):
    return b[0:1, :]


def _tc1_body(hvs, he, hvd, w1a, w1b, w1c, b1, w2, b2, w3, b3, l8_ref, gm_ref):
    i = pl.program_id(0)
    x = _mm(hvs[...], w1a[...]) + _mm(he[...], w1b[...]) + _mm(hvd[...], w1c[...])
    x = jnp.maximum(x + _row(b1[...]), 0.0)
    x = jnp.maximum(_mm(x, w2[...]) + _row(b2[...]), 0.0)
    l8 = _mm(x, w3[...]) + _row(b3[...])
    l8_ref[...] = l8
    bm = jnp.full((8, 128), jnp.max(l8), jnp.float32)

    @pl.when(i == 0)
    def _():
        gm_ref[...] = bm

    @pl.when(i > 0)
    def _():
        gm_ref[...] = jnp.maximum(gm_ref[...], bm)


def _edge_logits(gath, he, w1a, w1b, w1c, b1, w2, b2, w3, b3):
    eb = lambda i: (i, 0)
    eb2 = lambda i: (i + EGRID, 0)
    cb = lambda i: (0, 0)
    return pl.pallas_call(
        _tc1_body,
        grid=(EGRID,),
        in_specs=[
            pl.BlockSpec((EBLK, D), eb),
            pl.BlockSpec((EBLK, D), eb),
            pl.BlockSpec((EBLK, D), eb2),
            pl.BlockSpec((D, D), cb),
            pl.BlockSpec((D, D), cb),
            pl.BlockSpec((D, D), cb),
            pl.BlockSpec((8, D), cb),
            pl.BlockSpec((D, D), cb),
            pl.BlockSpec((8, D), cb),
            pl.BlockSpec((D, 8), cb),
            pl.BlockSpec((8, 8), cb),
        ],
        out_specs=[
            pl.BlockSpec((EBLK, 8), eb),
            pl.BlockSpec((8, 128), cb),
        ],
        out_shape=[
            jax.ShapeDtypeStruct((E, 8), jnp.float32),
            jax.ShapeDtypeStruct((8, 128), jnp.float32),
        ],
    )(gath, he, gath, w1a, w1b, w1c, b1, w2, b2, w3, b3)


def _tc2_body(l8, he, hvd, gm, wva, wvb, bv1, wv2, bv2, wv3, bv3, s128,
              numv_ref, exb_ref):
    m = gm[0, 0]
    ex8 = jnp.exp(l8[...] - m)
    exb = _mm(ex8, s128[...])
    exb_ref[...] = exb
    x = _gelu(_mm(he[...], wva[...]) + _mm(hvd[...], wvb[...]) + _row(bv1[...]))
    x = _gelu(_mm(x, wv2[...]) + _row(bv2[...]))
    v = _mm(x, wv3[...]) + _row(bv3[...])
    numv_ref[...] = exb * v


def _edge_payload(l8, he, gath, gm, wva, wvb, bv1, wv2, bv2, wv3, bv3, s128):
    eb = lambda i: (i, 0)
    eb2 = lambda i: (i + EGRID, 0)
    cb = lambda i: (0, 0)
    return pl.pallas_call(
        _tc2_body,
        grid=(EGRID,),
        in_specs=[
            pl.BlockSpec((EBLK, 8), eb),
            pl.BlockSpec((EBLK, D), eb),
            pl.BlockSpec((EBLK, D), eb2),
            pl.BlockSpec((8, 128), cb),
            pl.BlockSpec((D, D), cb),
            pl.BlockSpec((D, D), cb),
            pl.BlockSpec((8, D), cb),
            pl.BlockSpec((D, D), cb),
            pl.BlockSpec((8, D), cb),
            pl.BlockSpec((D, D), cb),
            pl.BlockSpec((8, D), cb),
            pl.BlockSpec((8, 128), cb),
        ],
        out_specs=[
            pl.BlockSpec((EBLK, D), eb),
            pl.BlockSpec((EBLK, D), eb),
        ],
        out_shape=[
            jax.ShapeDtypeStruct((E, D), jnp.float32),
            jax.ShapeDtypeStruct((E, D), jnp.float32),
        ],
    )(l8, he, gath, gm, wva, wvb, bv1, wv2, bv2, wv3, bv3, s128)


def _node_fused_body(n0, d0, hv, wot, g0, b0, w1t, bb1, w2t, bb2, g1, b1,
                     out_ref):
    num = n0[...]
    dden = d0[...]
    pos = dden > 0.0
    hagg = jnp.where(pos, num, 0.0) / jnp.where(pos, dden, 1.0)
    x1 = hv[...] + _mm(hagg, wot[...])
    m = (jnp.sum(x1, axis=0) * (1.0 / N))[None, :]
    v = (jnp.sum(x1 * x1, axis=0) * (1.0 / N))[None, :] - m * m
    xn = (x1 - m) * jax.lax.rsqrt(v + 1e-5) * _row(g0[...]) + _row(b0[...])
    h = jnp.maximum(_mm(xn, w1t[...]) + _row(bb1[...]), 0.0)
    x2 = xn + _mm(h, w2t[...]) + _row(bb2[...])
    m2 = (jnp.sum(x2, axis=0) * (1.0 / N))[None, :]
    v2 = (jnp.sum(x2 * x2, axis=0) * (1.0 / N))[None, :] - m2 * m2
    out_ref[...] = (x2 - m2) * jax.lax.rsqrt(v2 + 1e-5) * _row(g1[...]) + _row(b1[...])


def _node_fused(onum, oden, hv, wot, g0, b0, w1t, bb1, w2t, bb2, g1, b1):
    return pl.pallas_call(
        _node_fused_body,
        out_shape=jax.ShapeDtypeStruct((N, D), jnp.float32),
    )(onum, oden, hv, wot, g0, b0, w1t, bb1, w2t, bb2, g1, b1)


def _bn_body(nrows, x, s, q, g, b, out_ref):
    m = _row(s[...]) * (1.0 / nrows)
    v = _row(q[...]) * (1.0 / nrows) - m * m
    inv = jax.lax.rsqrt(v + 1e-5)
    out_ref[...] = (x[...] - m) * inv * _row(g[...]) + _row(b[...])


def _bn_apply(x, s, q, g, b, blk):
    nrows, _ = x.shape
    nb = lambda i: (i, 0)
    cb = lambda i: (0, 0)
    return pl.pallas_call(
        functools.partial(_bn_body, nrows),
        grid=(nrows // blk,),
        in_specs=[
            pl.BlockSpec((blk, D), nb),
            pl.BlockSpec((8, 128), cb),
            pl.BlockSpec((8, 128), cb),
            pl.BlockSpec((8, D), cb),
            pl.BlockSpec((8, D), cb),
        ],
        out_specs=pl.BlockSpec((blk, D), nb),
        out_shape=jax.ShapeDtypeStruct((nrows, D), jnp.float32),
    )(x, s, q, g, b)


def _tc6_body(hvs, he, hvd, w1a, w1b, w1c, b1, w2, b2, w3, b3,
              xe_ref, s_ref, q_ref):
    i = pl.program_id(0)
    x = _mm(hvs[...], w1a[...]) + _mm(he[...], w1b[...]) + _mm(hvd[...], w1c[...])
    x = _gelu(x + _row(b1[...]))
    x = _gelu(_mm(x, w2[...]) + _row(b2[...]))
    msg = _mm(x, w3[...]) + _row(b3[...])
    xe = he[...] + msg
    xe_ref[...] = xe
    s = jnp.broadcast_to(jnp.sum(xe, axis=0)[None, :], (8, 128))
    q = jnp.broadcast_to(jnp.sum(xe * xe, axis=0)[None, :], (8, 128))

    @pl.when(i == 0)
    def _():
        s_ref[...] = s
        q_ref[...] = q

    @pl.when(i > 0)
    def _():
        s_ref[...] += s
        q_ref[...] += q


def _edge_mlp(gath, he, w1a, w1b, w1c, b1, w2, b2, w3, b3):
    eb = lambda i: (i, 0)
    eb2 = lambda i: (i + EGRID, 0)
    cb = lambda i: (0, 0)
    return pl.pallas_call(
        _tc6_body,
        grid=(EGRID,),
        in_specs=[
            pl.BlockSpec((EBLK, D), eb),
            pl.BlockSpec((EBLK, D), eb),
            pl.BlockSpec((EBLK, D), eb2),
            pl.BlockSpec((D, D), cb),
            pl.BlockSpec((D, D), cb),
            pl.BlockSpec((D, D), cb),
            pl.BlockSpec((8, D), cb),
            pl.BlockSpec((D, D), cb),
            pl.BlockSpec((8, D), cb),
            pl.BlockSpec((D, D), cb),
            pl.BlockSpec((8, D), cb),
        ],
        out_specs=[
            pl.BlockSpec((EBLK, D), eb),
            pl.BlockSpec((8, 128), cb),
            pl.BlockSpec((8, 128), cb),
        ],
        out_shape=[
            jax.ShapeDtypeStruct((E, D), jnp.float32),
            jax.ShapeDtypeStruct((8, 128), jnp.float32),
            jax.ShapeDtypeStruct((8, 128), jnp.float32),
        ],
    )(gath, he, gath, w1a, w1b, w1c, b1, w2, b2, w3, b3)


def _bc8(b):
    return jnp.broadcast_to(b[None, :], (8, b.shape[0])).astype(jnp.float32)


def kernel(h_V, h_E, edge_idx, batch_id, params):
    p = params
    src = edge_idx[0]
    dst = edge_idx[1]

    allidx = jnp.concatenate([src, dst])
    gath = _sc_gather(h_V, allidx)

    sc = 1.0 / np.sqrt(DH)
    w3p = jnp.zeros((D, 8), jnp.float32).at[:, :H].set(p['bias_w3'].T * sc)
    b3p = jnp.full((8,), NEG, jnp.float32).at[:H].set(p['bias_b3'] * sc)
    l8, gm = _edge_logits(
        gath, h_E,
        p['bias_w1'][:, :D].T, p['bias_w1'][:, D:2 * D].T, p['bias_w1'][:, 2 * D:].T,
        _bc8(p['bias_b1']), p['bias_w2'].T, _bc8(p['bias_b2']), w3p, _bc8(b3p))

    s128 = np.zeros((8, 128), np.float32)
    for h in range(H):
        s128[h, h * DH:(h + 1) * DH] = 1.0
    numv, exb = _edge_payload(
        l8, h_E, gath, gm,
        p['wv_w1'][:, :D].T, p['wv_w1'][:, D:].T, _bc8(p['wv_b1']),
        p['wv_w2'].T, _bc8(p['wv_b2']), p['wv_w3'].T, _bc8(p['wv_b3']),
        jnp.asarray(s128))

    onum, oden = _sc_scatter(numv, exb, src, jnp.zeros((N, D), jnp.float32))

    h_V2 = _node_fused(onum, oden, h_V, p['wo'].T,
                       _bc8(p['bn0_g']), _bc8(p['bn0_b']),
                       p['dense_w1'].T, _bc8(p['dense_b1']),
                       p['dense_w2'].T, _bc8(p['dense_b2']),
                       _bc8(p['bn1_g']), _bc8(p['bn1_b']))

    gath2 = _sc_gather(h_V2, allidx)
    xe, se, qe = _edge_mlp(
        gath2, h_E,
        p['e_w11'][:, :D].T, p['e_w11'][:, D:2 * D].T, p['e_w11'][:, 2 * D:].T,
        _bc8(p['e_b11']), p['e_w12'].T, _bc8(p['e_b12']),
        p['e_w13'].T, _bc8(p['e_b13']))
    h_E2 = _bn_apply(xe, se, qe, _bc8(p['bne_g']), _bc8(p['bne_b']), EBLK)

    return h_V2, h_E2


# speedup vs baseline: 1.0627x; 1.0325x over previous
import functools

import jax
import jax.numpy as jnp
import numpy as np
from jax.experimental import pallas as pl
from jax.experimental.pallas import tpu as pltpu
from jax.experimental.pallas import tpu_sc as plsc

N = 10000
E = 160000
D = 128
H = 4
DH = D // H

NEG = -1e30

NC = 2
NS = 16
NW = NC * NS
CH = 40

EBLK = 1280
EGRID = E // EBLK
NBLK = 1000
NGRID = N // NBLK


def _mm(x, w):
    return jax.lax.dot_general(x.astype(jnp.bfloat16), w.astype(jnp.bfloat16),
                               (((1,), (0,)), ((), ())),
                               preferred_element_type=jnp.float32)


def _gelu(x):
    return 0.5 * x * (1.0 + jax.lax.erf(x * 0.7071067811865476))


def _sc_gather(table, idx1d):
    total = idx1d.shape[0]
    rows_per_w = total // NW
    K = 10
    groups = rows_per_w // (K * CH)
    PAIRS = groups // 2
    mesh = plsc.VectorSubcoreMesh(core_axis_name="c", subcore_axis_name="s")

    @functools.partial(
        pl.kernel,
        out_type=jax.ShapeDtypeStruct((total, D), jnp.float32),
        mesh=mesh,
        scratch_types=[pltpu.VMEM((CH,), jnp.int32)] * K + [
            pltpu.VMEM((K * CH, D), jnp.float32),
            pltpu.VMEM((K * CH, D), jnp.float32),
            pltpu.SemaphoreType.DMA,
            pltpu.SemaphoreType.DMA,
            pltpu.SemaphoreType.DMA,
            pltpu.SemaphoreType.DMA,
        ],
    )
    def k(table_hbm, idx_hbm, out_hbm, *rest):
        idx_bufs = rest[:K]
        rows_a, rows_b, sem_i, sem_g, sem_wa, sem_wb = rest[K:]
        rows = (rows_a, rows_b)
        sem_w = (sem_wa, sem_wb)
        cid = jax.lax.axis_index("c")
        sid = jax.lax.axis_index("s")
        wid = sid * NC + cid
        base = wid * rows_per_w

        def drain(buf):
            pltpu.make_async_copy(rows[buf],
                                  out_hbm.at[pl.ds(base, K * CH)],
                                  sem_w[buf]).wait()

        def do_group(g, buf):
            g0 = base + g * K * CH
            descs = [
                pltpu.async_copy(idx_hbm.at[pl.ds(g0 + i * CH, CH)],
                                 idx_bufs[i], sem_i)
                for i in range(K)
            ]
            for d in descs:
                d.wait()
            gds = [
                pltpu.async_copy(table_hbm.at[idx_bufs[i]],
                                 rows[buf].at[pl.ds(i * CH, CH)], sem_g)
                for i in range(K)
            ]
            for d in gds:
                d.wait()
            pltpu.async_copy(rows[buf], out_hbm.at[pl.ds(g0, K * CH)],
                             sem_w[buf])

        @pl.loop(0, PAIRS)
        def _(t):
            for buf in (0, 1):
                @pl.when(t > 0)
                def _():
                    drain(buf)

                do_group(2 * t + buf, buf)

        drain(0)
        do_group(groups - 1, 0)
        drain(0)
        drain(1)

    return k(table, idx1d)


def _sc_scatter(numv, exb, sidx, zn):
    e_per_tile = E // NS
    chunks_per_tile = e_per_tile // CH
    RC = 16
    KS = 5
    mesh = plsc.VectorSubcoreMesh(core_axis_name="c", subcore_axis_name="s")

    @functools.partial(
        pl.kernel,
        out_type=(jax.ShapeDtypeStruct((N, D), jnp.float32),
                  jax.ShapeDtypeStruct((N, D), jnp.float32)),
        mesh=mesh,
        scratch_types=[pltpu.VMEM((CH,), jnp.int32)] * KS + [
            pltpu.VMEM((CH, D), jnp.float32)] * KS + [
            pltpu.VMEM((CH, D), jnp.float32),
            pltpu.VMEM_SHARED((N, D), jnp.float32),
            pltpu.SemaphoreType.DMA,
            pltpu.SemaphoreType.DMA,
        ],
    )
    def k(numv_hbm, exb_hbm, idx_hbm, zn_hbm, onum_hbm, oden_hbm, *rest):
        idx_bufs = rest[:KS]
        pay_bufs = rest[KS:2 * KS]
        stage_v, acc, sem_l, sem_s = rest[2 * KS:]
        cid = jax.lax.axis_index("c")
        sid = jax.lax.axis_index("s")
        r0 = sid * RC * CH
        e_base = sid * e_per_tile
        groups = chunks_per_tile // KS

        pltpu.sync_copy(zn_hbm.at[pl.ds(0, CH)], stage_v)

        @pl.loop(0, RC)
        def _(j):
            @pl.when(r0 + j * CH < N)
            def _():
                pltpu.sync_copy(stage_v, acc.at[pl.ds(r0 + j * CH, CH)])

        plsc.subcore_barrier()

        def scatter_pass(src_hbm):
            @pl.loop(0, groups)
            def _(g):
                g0 = e_base + g * KS * CH
                lds = [pltpu.async_copy(idx_hbm.at[pl.ds(g0 + i * CH, CH)],
                                        idx_bufs[i], sem_l)
                       for i in range(KS)]
                lds += [pltpu.async_copy(src_hbm.at[pl.ds(g0 + i * CH, CH)],
                                         pay_bufs[i], sem_l)
                        for i in range(KS)]
                for d in lds:
                    d.wait()
                sds = [pltpu.async_copy(pay_bufs[i], acc.at[idx_bufs[i]],
                                        sem_s, add=True)
                       for i in range(KS)]
                for d in sds:
                    d.wait()

        @pl.when(cid == 0)
        def _():
            scatter_pass(numv_hbm)

        @pl.when(cid == 1)
        def _():
            scatter_pass(exb_hbm)

        plsc.subcore_barrier()

        def copy_out(dst_hbm):
            @pl.loop(0, RC)
            def _(j):
                c0 = r0 + j * CH

                @pl.when(c0 < N)
                def _():
                    pltpu.sync_copy(acc.at[pl.ds(c0, CH)], stage_v)
                    pltpu.sync_copy(stage_v, dst_hbm.at[pl.ds(c0, CH)])

        @pl.when(cid == 0)
        def _():
            copy_out(onum_hbm)

        @pl.when(cid == 1)
        def _():
            copy_out(oden_hbm)

    return k(numv, exb, sidx, zn)


def _row(b):
    return b[0:1, :]


def _tc1_body(hvs, he, hvd, w1a, w1b, w1c, b1, w2, b2, w3, b3, l8_ref, gm_ref):
    i = pl.program_id(0)
    x = _mm(hvs[...], w1a[...]) + _mm(he[...], w1b[...]) + _mm(hvd[...], w1c[...])
    x = jnp.maximum(x + _row(b1[...]), 0.0)
    x = jnp.maximum(_mm(x, w2[...]) + _row(b2[...]), 0.0)
    l8 = _mm(x, w3[...]) + _row(b3[...])
    l8_ref[...] = l8
    bm = jnp.full((8, 128), jnp.max(l8), jnp.float32)

    @pl.when(i == 0)
    def _():
        gm_ref[...] = bm

    @pl.when(i > 0)
    def _():
        gm_ref[...] = jnp.maximum(gm_ref[...], bm)


def _edge_logits(gath, he, w1a, w1b, w1c, b1, w2, b2, w3, b3):
    eb = lambda i: (i, 0)
    eb2 = lambda i: (i + EGRID, 0)
    cb = lambda i: (0, 0)
    return pl.pallas_call(
        _tc1_body,
        grid=(EGRID,),
        in_specs=[
            pl.BlockSpec((EBLK, D), eb),
            pl.BlockSpec((EBLK, D), eb),
            pl.BlockSpec((EBLK, D), eb2),
            pl.BlockSpec((D, D), cb),
            pl.BlockSpec((D, D), cb),
            pl.BlockSpec((D, D), cb),
            pl.BlockSpec((8, D), cb),
            pl.BlockSpec((D, D), cb),
            pl.BlockSpec((8, D), cb),
            pl.BlockSpec((D, 8), cb),
            pl.BlockSpec((8, 8), cb),
        ],
        out_specs=[
            pl.BlockSpec((EBLK, 8), eb),
            pl.BlockSpec((8, 128), cb),
        ],
        out_shape=[
            jax.ShapeDtypeStruct((E, 8), jnp.float32),
            jax.ShapeDtypeStruct((8, 128), jnp.float32),
        ],
    )(gath, he, gath, w1a, w1b, w1c, b1, w2, b2, w3, b3)


def _tc2_body(l8, he, hvd, gm, wva, wvb, bv1, wv2, bv2, wv3, bv3, s128,
              numv_ref, exb_ref):
    m = gm[0, 0]
    ex8 = jnp.exp(l8[...] - m)
    exb = _mm(ex8, s128[...])
    exb_ref[...] = exb
    x = _gelu(_mm(he[...], wva[...]) + _mm(hvd[...], wvb[...]) + _row(bv1[...]))
    x = _gelu(_mm(x, wv2[...]) + _row(bv2[...]))
    v = _mm(x, wv3[...]) + _row(bv3[...])
    numv_ref[...] = exb * v


def _edge_payload(l8, he, gath, gm, wva, wvb, bv1, wv2, bv2, wv3, bv3, s128):
    eb = lambda i: (i, 0)
    eb2 = lambda i: (i + EGRID, 0)
    cb = lambda i: (0, 0)
    return pl.pallas_call(
        _tc2_body,
        grid=(EGRID,),
        in_specs=[
            pl.BlockSpec((EBLK, 8), eb),
            pl.BlockSpec((EBLK, D), eb),
            pl.BlockSpec((EBLK, D), eb2),
            pl.BlockSpec((8, 128), cb),
            pl.BlockSpec((D, D), cb),
            pl.BlockSpec((D, D), cb),
            pl.BlockSpec((8, D), cb),
            pl.BlockSpec((D, D), cb),
            pl.BlockSpec((8, D), cb),
            pl.BlockSpec((D, D), cb),
            pl.BlockSpec((8, D), cb),
            pl.BlockSpec((8, 128), cb),
        ],
        out_specs=[
            pl.BlockSpec((EBLK, D), eb),
            pl.BlockSpec((EBLK, D), eb),
        ],
        out_shape=[
            jax.ShapeDtypeStruct((E, D), jnp.float32),
            jax.ShapeDtypeStruct((E, D), jnp.float32),
        ],
    )(l8, he, gath, gm, wva, wvb, bv1, wv2, bv2, wv3, bv3, s128)


def _node_fused_body(n0, d0, hv, wot, g0, b0, w1t, bb1, w2t, bb2, g1, b1,
                     out_ref):
    num = n0[...]
    dden = d0[...]
    pos = dden > 0.0
    hagg = jnp.where(pos, num, 0.0) / jnp.where(pos, dden, 1.0)
    x1 = hv[...] + _mm(hagg, wot[...])
    m = (jnp.sum(x1, axis=0) * (1.0 / N))[None, :]
    v = (jnp.sum(x1 * x1, axis=0) * (1.0 / N))[None, :] - m * m
    xn = (x1 - m) * jax.lax.rsqrt(v + 1e-5) * _row(g0[...]) + _row(b0[...])
    h = jnp.maximum(_mm(xn, w1t[...]) + _row(bb1[...]), 0.0)
    x2 = xn + _mm(h, w2t[...]) + _row(bb2[...])
    m2 = (jnp.sum(x2, axis=0) * (1.0 / N))[None, :]
    v2 = (jnp.sum(x2 * x2, axis=0) * (1.0 / N))[None, :] - m2 * m2
    out_ref[...] = (x2 - m2) * jax.lax.rsqrt(v2 + 1e-5) * _row(g1[...]) + _row(b1[...])


def _node_fused(onum, oden, hv, wot, g0, b0, w1t, bb1, w2t, bb2, g1, b1):
    return pl.pallas_call(
        _node_fused_body,
        out_shape=jax.ShapeDtypeStruct((N, D), jnp.float32),
    )(onum, oden, hv, wot, g0, b0, w1t, bb1, w2t, bb2, g1, b1)


def _bn_body(nrows, x, s, q, g, b, out_ref):
    m = _row(s[...]) * (1.0 / nrows)
    v = _row(q[...]) * (1.0 / nrows) - m * m
    inv = jax.lax.rsqrt(v + 1e-5)
    out_ref[...] = (x[...] - m) * inv * _row(g[...]) + _row(b[...])


def _bn_apply(x, s, q, g, b, blk):
    nrows, _ = x.shape
    nb = lambda i: (i, 0)
    cb = lambda i: (0, 0)
    return pl.pallas_call(
        functools.partial(_bn_body, nrows),
        grid=(nrows // blk,),
        in_specs=[
            pl.BlockSpec((blk, D), nb),
            pl.BlockSpec((8, 128), cb),
            pl.BlockSpec((8, 128), cb),
            pl.BlockSpec((8, D), cb),
            pl.BlockSpec((8, D), cb),
        ],
        out_specs=pl.BlockSpec((blk, D), nb),
        out_shape=jax.ShapeDtypeStruct((nrows, D), jnp.float32),
    )(x, s, q, g, b)


def _tc6_body(hvs, he, hvd, w1a, w1b, w1c, b1, w2, b2, w3, b3,
              xe_ref, s_ref, q_ref):
    i = pl.program_id(0)
    x = _mm(hvs[...], w1a[...]) + _mm(he[...], w1b[...]) + _mm(hvd[...], w1c[...])
    x = _gelu(x + _row(b1[...]))
    x = _gelu(_mm(x, w2[...]) + _row(b2[...]))
    msg = _mm(x, w3[...]) + _row(b3[...])
    xe = he[...] + msg
    xe_ref[...] = xe
    s = jnp.broadcast_to(jnp.sum(xe, axis=0)[None, :], (8, 128))
    q = jnp.broadcast_to(jnp.sum(xe * xe, axis=0)[None, :], (8, 128))

    @pl.when(i == 0)
    def _():
        s_ref[...] = s
        q_ref[...] = q

    @pl.when(i > 0)
    def _():
        s_ref[...] += s
        q_ref[...] += q


def _edge_mlp(gath, he, w1a, w1b, w1c, b1, w2, b2, w3, b3):
    eb = lambda i: (i, 0)
    eb2 = lambda i: (i + EGRID, 0)
    cb = lambda i: (0, 0)
    return pl.pallas_call(
        _tc6_body,
        grid=(EGRID,),
        in_specs=[
            pl.BlockSpec((EBLK, D), eb),
            pl.BlockSpec((EBLK, D), eb),
            pl.BlockSpec((EBLK, D), eb2),
            pl.BlockSpec((D, D), cb),
            pl.BlockSpec((D, D), cb),
            pl.BlockSpec((D, D), cb),
            pl.BlockSpec((8, D), cb),
            pl.BlockSpec((D, D), cb),
            pl.BlockSpec((8, D), cb),
            pl.BlockSpec((D, D), cb),
            pl.BlockSpec((8, D), cb),
        ],
        out_specs=[
            pl.BlockSpec((EBLK, D), eb),
            pl.BlockSpec((8, 128), cb),
            pl.BlockSpec((8, 128), cb),
        ],
        out_shape=[
            jax.ShapeDtypeStruct((E, D), jnp.float32),
            jax.ShapeDtypeStruct((8, 128), jnp.float32),
            jax.ShapeDtypeStruct((8, 128), jnp.float32),
        ],
    )(gath, he, gath, w1a, w1b, w1c, b1, w2, b2, w3, b3)


def _bc8(b):
    return jnp.broadcast_to(b[None, :], (8, b.shape[0])).astype(jnp.float32)


def kernel(h_V, h_E, edge_idx, batch_id, params):
    p = params
    src = edge_idx[0]
    dst = edge_idx[1]

    allidx = jnp.concatenate([src, dst])
    gath = _sc_gather(h_V, allidx)

    sc = 1.0 / np.sqrt(DH)
    w3p = jnp.zeros((D, 8), jnp.float32).at[:, :H].set(p['bias_w3'].T * sc)
    b3p = jnp.full((8,), NEG, jnp.float32).at[:H].set(p['bias_b3'] * sc)
    l8, gm = _edge_logits(
        gath, h_E,
        p['bias_w1'][:, :D].T, p['bias_w1'][:, D:2 * D].T, p['bias_w1'][:, 2 * D:].T,
        _bc8(p['bias_b1']), p['bias_w2'].T, _bc8(p['bias_b2']), w3p, _bc8(b3p))

    s128 = np.zeros((8, 128), np.float32)
    for h in range(H):
        s128[h, h * DH:(h + 1) * DH] = 1.0
    numv, exb = _edge_payload(
        l8, h_E, gath, gm,
        p['wv_w1'][:, :D].T, p['wv_w1'][:, D:].T, _bc8(p['wv_b1']),
        p['wv_w2'].T, _bc8(p['wv_b2']), p['wv_w3'].T, _bc8(p['wv_b3']),
        jnp.asarray(s128))

    onum, oden = _sc_scatter(numv, exb, src, jnp.zeros((N, D), jnp.float32))

    h_V2 = _node_fused(onum, oden, h_V, p['wo'].T,
                       _bc8(p['bn0_g']), _bc8(p['bn0_b']),
                       p['dense_w1'].T, _bc8(p['dense_b1']),
                       p['dense_w2'].T, _bc8(p['dense_b2']),
                       _bc8(p['bn1_g']), _bc8(p['bn1_b']))

    gath2 = _sc_gather(h_V2, allidx)
    xe, se, qe = _edge_mlp(
        gath2, h_E,
        p['e_w11'][:, :D].T, p['e_w11'][:, D:2 * D].T, p['e_w11'][:, 2 * D:].T,
        _bc8(p['e_b11']), p['e_w12'].T, _bc8(p['e_b12']),
        p['e_w13'].T, _bc8(p['e_b13']))
    h_E2 = _bn_apply(xe, se, qe, _bc8(p['bne_g']), _bc8(p['bne_b']), EBLK)

    return h_V2, h_E2
